# initial kernel scaffold (unmeasured)
import functools

import jax
import jax.numpy as jnp
from jax import lax
from jax.experimental import pallas as pl
from jax.experimental.pallas import tpu as pltpu

N_DEV = 4
SQ = 2048
SKV = 2048
D_MODEL = 1024
HQ_PER = 8
DH = 128
DQ_PER = HQ_PER * DH
SCALE = 0.08838834764831843
NEG = -30000.0

ORDER = (0, 1, 3, 2)


def kernel(x, Wq, K_ext, V_ext, Wo):
    xb = x[0].astype(jnp.bfloat16)
    wq = Wq.astype(jnp.bfloat16)
    wo = Wo.astype(jnp.bfloat16)

    def body(
        x_ref,
        wq_ref,
        k_hbm,
        v_hbm,
        wo_ref,
        out_ref,
        w_buf,
        k_scr,
        v_scr,
        bias_scr,
        send_sems,
        recv_sems,
        k_sems,
        v_sems,
    ):
        my_i = lax.axis_index("i")

        barrier_sem = pltpu.get_barrier_semaphore()
        for g in (1, 2, 3):
            pl.semaphore_signal(
                barrier_sem,
                inc=1,
                device_id=((my_i + g) % N_DEV,),
                device_id_type=pl.DeviceIdType.MESH,
            )
        pl.semaphore_wait(barrier_sem, 3)

        w_buf[my_i, 0] = wq_ref[...]
        w_buf[my_i, 1] = wo_ref[...]
        sends = []
        for g in (1, 2, 3):
            rdma = pltpu.make_async_remote_copy(
                src_ref=w_buf.at[my_i],
                dst_ref=w_buf.at[my_i],
                send_sem=send_sems.at[g],
                recv_sem=recv_sems.at[4 - g],
                device_id=((my_i + g) % N_DEV,),
                device_id_type=pl.DeviceIdType.MESH,
            )
            rdma.start()
            sends.append(rdma)

        qi = lax.broadcasted_iota(jnp.int32, (SQ, SKV), 0)
        ki = lax.broadcasted_iota(jnp.int32, (SQ, SKV), 1)
        mask = (jnp.abs(qi - ki) <= 128) | (ki < 32) | (qi < 32)
        bias_scr[...] = jnp.where(mask, 0.0, NEG).astype(jnp.bfloat16)

        x_bf = x_ref[...]

        for j in range(N_DEV):
            d = ORDER[j]
            o = (my_i + d) % N_DEV

            kops = []
            vops = []
            for h in range(HQ_PER):
                head = o * HQ_PER + h
                kop = pltpu.make_async_copy(
                    k_hbm.at[my_i, :, head, :], k_scr.at[h], k_sems.at[h]
                )
                vop = pltpu.make_async_copy(
                    v_hbm.at[my_i, :, head, :], v_scr.at[h], v_sems.at[h]
                )
                kop.start()
                vop.start()
                kops.append(kop)
                vops.append(vop)

            if d != 0:
                recv = pltpu.make_async_remote_copy(
                    src_ref=w_buf.at[o],
                    dst_ref=w_buf.at[o],
                    send_sem=send_sems.at[0],
                    recv_sem=recv_sems.at[d],
                    device_id=(my_i,),
                    device_id_type=pl.DeviceIdType.MESH,
                )
                recv.wait_recv()

            q = lax.dot_general(
                x_bf,
                w_buf[o, 0],
                (((1,), (0,)), ((), ())),
                preferred_element_type=jnp.float32,
            )
            q = (q * SCALE).astype(jnp.bfloat16)

            ctx_parts = []
            for h in range(HQ_PER):
                kops[h].wait()
                vops[h].wait()
                k_bf = k_scr[h].astype(jnp.bfloat16)
                v_bf = v_scr[h].astype(jnp.bfloat16)
                q_h = q[:, h * DH : (h + 1) * DH]
                s = lax.dot_general(
                    q_h,
                    k_bf,
                    (((1,), (1,)), ((), ())),
                    preferred_element_type=jnp.float32,
                )
                s = s + bias_scr[...]
                m = jnp.max(s, axis=1, keepdims=True)
                e = jnp.exp(s - m)
                denom = jnp.sum(e, axis=1, keepdims=True)
                w = (e / denom).astype(jnp.bfloat16)
                ctx_parts.append(
                    lax.dot_general(
                        w,
                        v_bf,
                        (((1,), (0,)), ((), ())),
                        preferred_element_type=jnp.float32,
                    )
                )
            ctx = jnp.concatenate(ctx_parts, axis=1).astype(jnp.bfloat16)
            part = lax.dot_general(
                ctx,
                w_buf[o, 1],
                (((1,), (0,)), ((), ())),
                preferred_element_type=jnp.float32,
            )
            if j == 0:
                out_ref[0] = part
            else:
                out_ref[0] += part

        for s_ in sends:
            s_.wait_send()

        @functools.partial(pl.run_scoped, sem=pltpu.SemaphoreType.REGULAR)
        def _(sem):
            for g in (1, 2, 3):
                pl.semaphore_signal(
                    sem,
                    inc=1,
                    device_id=((my_i + g) % N_DEV,),
                    device_id_type=pl.DeviceIdType.MESH,
                )
            pl.semaphore_wait(sem, 3)

    return pl.pallas_call(
        body,
        out_shape=jax.ShapeDtypeStruct((1, SQ, D_MODEL), jnp.float32),
        in_specs=[
            pl.BlockSpec(memory_space=pltpu.VMEM),
            pl.BlockSpec(memory_space=pltpu.VMEM),
            pl.BlockSpec(memory_space=pltpu.ANY),
            pl.BlockSpec(memory_space=pltpu.ANY),
            pl.BlockSpec(memory_space=pltpu.VMEM),
        ],
        out_specs=pl.BlockSpec(memory_space=pltpu.VMEM),
        scratch_shapes=[
            pltpu.VMEM((N_DEV, 2, D_MODEL, DQ_PER), jnp.bfloat16),
            pltpu.VMEM((HQ_PER, SKV, DH), jnp.float32),
            pltpu.VMEM((HQ_PER, SKV, DH), jnp.float32),
            pltpu.VMEM((SQ, SKV), jnp.bfloat16),
            pltpu.SemaphoreType.DMA((N_DEV,)),
            pltpu.SemaphoreType.DMA((N_DEV,)),
            pltpu.SemaphoreType.DMA((HQ_PER,)),
            pltpu.SemaphoreType.DMA((HQ_PER,)),
        ],
        compiler_params=pltpu.CompilerParams(collective_id=0),
    )(xb, wq, K_ext, V_ext, wo)


# baseline (device time: 440223 ns/iter reference)
import functools

import jax
import jax.numpy as jnp
from jax import lax
from jax.experimental import pallas as pl
from jax.experimental.pallas import tpu as pltpu

N_DEV = 4
SQ = 2048
SKV = 2048
D_MODEL = 1024
HQ_PER = 8
DH = 128
DQ_PER = HQ_PER * DH
SCALE = 0.08838834764831843
NEG = -30000.0
QT = 256

ORDER = (0, 1, 3, 2)


def kernel(x, Wq, K_ext, V_ext, Wo):
    xb = x[0].astype(jnp.bfloat16)
    wq = Wq.astype(jnp.bfloat16)
    wo = Wo.astype(jnp.bfloat16)
    my = lax.axis_index("i")
    k_my = (
        lax.dynamic_index_in_dim(K_ext, my, 0, keepdims=False)
        .astype(jnp.bfloat16)
        .transpose(1, 0, 2)
    )
    v_my = (
        lax.dynamic_index_in_dim(V_ext, my, 0, keepdims=False)
        .astype(jnp.bfloat16)
        .transpose(1, 0, 2)
    )

    def body(
        x_ref,
        wq_ref,
        k_hbm,
        v_hbm,
        wo_ref,
        out_ref,
        wq_buf,
        wo_buf,
        k_scr,
        v_scr,
        bias_scr,
        sendq_sems,
        sendo_sems,
        recvq_sems,
        recvo_sems,
        k_sems,
        v_sems,
    ):
        my_i = lax.axis_index("i")

        barrier_sem = pltpu.get_barrier_semaphore()
        for g in (1, 2, 3):
            pl.semaphore_signal(
                barrier_sem,
                inc=1,
                device_id=((my_i + g) % N_DEV,),
                device_id_type=pl.DeviceIdType.MESH,
            )
        pl.semaphore_wait(barrier_sem, 3)

        wq_buf[my_i] = wq_ref[...]
        wo_buf[my_i] = wo_ref[...]
        sends = []
        for g in (1, 2, 3):
            for buf, ssems, rsems in (
                (wq_buf, sendq_sems, recvq_sems),
                (wo_buf, sendo_sems, recvo_sems),
            ):
                rdma = pltpu.make_async_remote_copy(
                    src_ref=buf.at[my_i],
                    dst_ref=buf.at[my_i],
                    send_sem=ssems.at[g],
                    recv_sem=rsems.at[4 - g],
                    device_id=((my_i + g) % N_DEV,),
                    device_id_type=pl.DeviceIdType.MESH,
                )
                rdma.start()
                sends.append(rdma)

        rows = 512
        for c in range(SQ // rows):
            qi = lax.broadcasted_iota(jnp.int32, (rows, SKV), 0) + c * rows
            ki = lax.broadcasted_iota(jnp.int32, (rows, SKV), 1)
            mask = (jnp.abs(qi - ki) <= 128) | (ki < 32) | (qi < 32)
            bias_scr[c * rows : (c + 1) * rows, :] = jnp.where(
                mask, 0.0, NEG
            ).astype(jnp.bfloat16)

        for j in range(N_DEV):
            d = ORDER[j]
            o = (my_i + d) % N_DEV

            ops = []
            kop = pltpu.make_async_copy(
                k_hbm.at[pl.ds(o * HQ_PER, HQ_PER)], k_scr, k_sems.at[0]
            )
            vop = pltpu.make_async_copy(
                v_hbm.at[pl.ds(o * HQ_PER, HQ_PER)], v_scr, v_sems.at[0]
            )
            kop.start()
            vop.start()
            ops.append(kop)
            ops.append(vop)

            if d != 0:
                for buf, rsems in ((wq_buf, recvq_sems), (wo_buf, recvo_sems)):
                    recv = pltpu.make_async_remote_copy(
                        src_ref=buf.at[o],
                        dst_ref=buf.at[o],
                        send_sem=sendq_sems.at[0],
                        recv_sem=rsems.at[d],
                        device_id=(my_i,),
                        device_id_type=pl.DeviceIdType.MESH,
                    )
                    recv.wait_recv()
            for op in ops:
                op.wait()

            def tile_step(qt, _):
                r0 = qt * QT
                x_t = x_ref[pl.ds(r0, QT), :]
                q_t = lax.dot_general(
                    x_t,
                    wq_buf[o],
                    (((1,), (0,)), ((), ())),
                    preferred_element_type=jnp.float32,
                )
                q_t = (q_t * SCALE).astype(jnp.bfloat16)
                bias_t = bias_scr[pl.ds(r0, QT), :]
                ctx_parts = []
                for h in range(HQ_PER):
                    s = lax.dot_general(
                        q_t[:, h * DH : (h + 1) * DH],
                        k_scr[h],
                        (((1,), (1,)), ((), ())),
                        preferred_element_type=jnp.float32,
                    )
                    s = s + bias_t
                    m = jnp.max(s, axis=1, keepdims=True)
                    e = jnp.exp(s - m)
                    denom = jnp.sum(e, axis=1, keepdims=True)
                    w = (e / denom).astype(jnp.bfloat16)
                    ctx_parts.append(
                        lax.dot_general(
                            w,
                            v_scr[h],
                            (((1,), (0,)), ((), ())),
                            preferred_element_type=jnp.float32,
                        ).astype(jnp.bfloat16)
                    )
                ctx = jnp.concatenate(ctx_parts, axis=1)
                part = lax.dot_general(
                    ctx,
                    wo_buf[o],
                    (((1,), (0,)), ((), ())),
                    preferred_element_type=jnp.float32,
                )
                if j == 0:
                    out_ref[0, pl.ds(r0, QT), :] = part
                else:
                    out_ref[0, pl.ds(r0, QT), :] += part
                return 0

            lax.fori_loop(0, SQ // QT, tile_step, 0)

        for s_ in sends:
            s_.wait_send()

        @functools.partial(pl.run_scoped, sem=pltpu.SemaphoreType.REGULAR)
        def _(sem):
            for g in (1, 2, 3):
                pl.semaphore_signal(
                    sem,
                    inc=1,
                    device_id=((my_i + g) % N_DEV,),
                    device_id_type=pl.DeviceIdType.MESH,
                )
            pl.semaphore_wait(sem, 3)

    return pl.pallas_call(
        body,
        out_shape=jax.ShapeDtypeStruct((1, SQ, D_MODEL), jnp.float32),
        in_specs=[
            pl.BlockSpec(memory_space=pltpu.MemorySpace.VMEM),
            pl.BlockSpec(memory_space=pltpu.MemorySpace.VMEM),
            pl.BlockSpec(memory_space=pl.ANY),
            pl.BlockSpec(memory_space=pl.ANY),
            pl.BlockSpec(memory_space=pltpu.MemorySpace.VMEM),
        ],
        out_specs=pl.BlockSpec(memory_space=pltpu.MemorySpace.VMEM),
        scratch_shapes=[
            pltpu.VMEM((N_DEV, D_MODEL, DQ_PER), jnp.bfloat16),
            pltpu.VMEM((N_DEV, DQ_PER, D_MODEL), jnp.bfloat16),
            pltpu.VMEM((HQ_PER, SKV, DH), jnp.bfloat16),
            pltpu.VMEM((HQ_PER, SKV, DH), jnp.bfloat16),
            pltpu.VMEM((SQ, SKV), jnp.bfloat16),
            pltpu.SemaphoreType.DMA((N_DEV,)),
            pltpu.SemaphoreType.DMA((N_DEV,)),
            pltpu.SemaphoreType.DMA((N_DEV,)),
            pltpu.SemaphoreType.DMA((N_DEV,)),
            pltpu.SemaphoreType.DMA((HQ_PER,)),
            pltpu.SemaphoreType.DMA((HQ_PER,)),
        ],
        compiler_params=pltpu.CompilerParams(
            collective_id=0,
            vmem_limit_bytes=110 * 1024 * 1024,
        ),
    )(xb, wq, k_my, v_my, wo)


# device time: 374559 ns/iter; 1.1753x vs baseline; 1.1753x over previous
import functools

import jax
import jax.numpy as jnp
from jax import lax
from jax.experimental import pallas as pl
from jax.experimental.pallas import tpu as pltpu

N_DEV = 4
SQ = 2048
SKV = 2048
D_MODEL = 1024
HQ_PER = 8
DH = 128
DQ_PER = HQ_PER * DH
SCALE = 0.08838834764831843
NEG = -30000.0
QT = 128
GW = 128
WIN = 384
KW = GW + WIN

ORDER = (0, 1, 3, 2)


def _softmax_ctx(s, v):
    m = jnp.max(s, axis=1, keepdims=True)
    e = jnp.exp(s - m)
    denom = jnp.sum(e, axis=1, keepdims=True)
    w = (e / denom).astype(jnp.bfloat16)
    return lax.dot_general(
        w, v, (((1,), (0,)), ((), ())), preferred_element_type=jnp.float32
    ).astype(jnp.bfloat16)


def kernel(x, Wq, K_ext, V_ext, Wo):
    xb = x[0].astype(jnp.bfloat16)
    wq = Wq.astype(jnp.bfloat16)
    wo = Wo.astype(jnp.bfloat16)
    my = lax.axis_index("i")
    k_my = (
        lax.dynamic_index_in_dim(K_ext, my, 0, keepdims=False)
        .astype(jnp.bfloat16)
        .transpose(1, 0, 2)
    )
    v_my = (
        lax.dynamic_index_in_dim(V_ext, my, 0, keepdims=False)
        .astype(jnp.bfloat16)
        .transpose(1, 0, 2)
    )

    def body(
        x_ref,
        wq_ref,
        k_hbm,
        v_hbm,
        wo_ref,
        out_ref,
        wq_buf,
        wo_buf,
        k_scr,
        v_scr,
        sendq_sems,
        sendo_sems,
        recvq_sems,
        recvo_sems,
        k_sems,
        v_sems,
    ):
        my_i = lax.axis_index("i")

        barrier_sem = pltpu.get_barrier_semaphore()
        for g in (1, 2, 3):
            pl.semaphore_signal(
                barrier_sem,
                inc=1,
                device_id=((my_i + g) % N_DEV,),
                device_id_type=pl.DeviceIdType.MESH,
            )
        pl.semaphore_wait(barrier_sem, 3)

        wq_buf[my_i] = wq_ref[...]
        wo_buf[my_i] = wo_ref[...]
        sends = []
        for g in (1, 2, 3):
            for buf, ssems, rsems in (
                (wq_buf, sendq_sems, recvq_sems),
                (wo_buf, sendo_sems, recvo_sems),
            ):
                rdma = pltpu.make_async_remote_copy(
                    src_ref=buf.at[my_i],
                    dst_ref=buf.at[my_i],
                    send_sem=ssems.at[g],
                    recv_sem=rsems.at[4 - g],
                    device_id=((my_i + g) % N_DEV,),
                    device_id_type=pl.DeviceIdType.MESH,
                )
                rdma.start()
                sends.append(rdma)

        def kv_dma(j, slot):
            og = (my_i + ORDER[j]) % N_DEV
            kop = pltpu.make_async_copy(
                k_hbm.at[pl.ds(og * HQ_PER, HQ_PER)], k_scr.at[slot], k_sems.at[slot]
            )
            vop = pltpu.make_async_copy(
                v_hbm.at[pl.ds(og * HQ_PER, HQ_PER)], v_scr.at[slot], v_sems.at[slot]
            )
            return kop, vop

        k0, v0 = kv_dma(0, 0)
        k0.start()
        v0.start()

        for j in range(N_DEV):
            d = ORDER[j]
            o = (my_i + d) % N_DEV
            slot = j % 2

            if j < N_DEV - 1:
                kn, vn = kv_dma(j + 1, 1 - slot)
                kn.start()
                vn.start()

            if d != 0:
                for buf, rsems in ((wq_buf, recvq_sems), (wo_buf, recvo_sems)):
                    recv = pltpu.make_async_remote_copy(
                        src_ref=buf.at[o],
                        dst_ref=buf.at[o],
                        send_sem=sendq_sems.at[0],
                        recv_sem=rsems.at[d],
                        device_id=(my_i,),
                        device_id_type=pl.DeviceIdType.MESH,
                    )
                    recv.wait_recv()
            kw_, vw_ = kv_dma(j, slot)
            kw_.wait()
            vw_.wait()

            x_t = x_ref[0:QT, :]
            q_t = lax.dot_general(
                x_t,
                wq_buf[o],
                (((1,), (0,)), ((), ())),
                preferred_element_type=jnp.float32,
            )
            q_t = (q_t * SCALE).astype(jnp.bfloat16)
            qi = lax.broadcasted_iota(jnp.int32, (QT, SKV), 0)
            ki = lax.broadcasted_iota(jnp.int32, (QT, SKV), 1)
            keep = (jnp.abs(qi - ki) <= 128) | (ki < 32) | (qi < 32)
            bias0 = jnp.where(keep, 0.0, NEG)
            ctx_parts = []
            for h in range(HQ_PER):
                s = lax.dot_general(
                    q_t[:, h * DH : (h + 1) * DH],
                    k_scr[slot, h],
                    (((1,), (1,)), ((), ())),
                    preferred_element_type=jnp.float32,
                )
                ctx_parts.append(_softmax_ctx(s + bias0, v_scr[slot, h]))
            ctx = jnp.concatenate(ctx_parts, axis=1)
            part = lax.dot_general(
                ctx,
                wo_buf[o],
                (((1,), (0,)), ((), ())),
                preferred_element_type=jnp.float32,
            )
            if j == 0:
                out_ref[0, 0:QT, :] = part
            else:
                out_ref[0, 0:QT, :] += part

            def tile_step(qt, _):
                r0 = qt * QT
                s0 = jnp.minimum(r0 - 128, SKV - WIN)
                x_tt = x_ref[pl.ds(r0, QT), :]
                q_tt = lax.dot_general(
                    x_tt,
                    wq_buf[o],
                    (((1,), (0,)), ((), ())),
                    preferred_element_type=jnp.float32,
                )
                q_tt = (q_tt * SCALE).astype(jnp.bfloat16)
                row = lax.broadcasted_iota(jnp.int32, (QT, KW), 0) + r0
                col = lax.broadcasted_iota(jnp.int32, (QT, KW), 1)
                in_glob = col < GW
                kiw = s0 + col - GW
                win_keep = (jnp.abs(row - kiw) <= 128) | (kiw < 32)
                keep_t = (in_glob & (col < 32) & (s0 >= GW)) | (
                    (~in_glob) & win_keep
                )
                bias_t = jnp.where(keep_t, 0.0, NEG)
                kcat = jnp.concatenate(
                    [k_scr[slot, :, 0:GW, :], k_scr[slot, :, pl.ds(s0, WIN), :]],
                    axis=1,
                )
                vcat = jnp.concatenate(
                    [v_scr[slot, :, 0:GW, :], v_scr[slot, :, pl.ds(s0, WIN), :]],
                    axis=1,
                )
                parts = []
                for h in range(HQ_PER):
                    s = lax.dot_general(
                        q_tt[:, h * DH : (h + 1) * DH],
                        kcat[h],
                        (((1,), (1,)), ((), ())),
                        preferred_element_type=jnp.float32,
                    )
                    parts.append(_softmax_ctx(s + bias_t, vcat[h]))
                ctx_t = jnp.concatenate(parts, axis=1)
                part_t = lax.dot_general(
                    ctx_t,
                    wo_buf[o],
                    (((1,), (0,)), ((), ())),
                    preferred_element_type=jnp.float32,
                )
                if j == 0:
                    out_ref[0, pl.ds(r0, QT), :] = part_t
                else:
                    out_ref[0, pl.ds(r0, QT), :] += part_t
                return 0

            lax.fori_loop(1, SQ // QT, tile_step, 0)

        for s_ in sends:
            s_.wait_send()

        @functools.partial(pl.run_scoped, sem=pltpu.SemaphoreType.REGULAR)
        def _(sem):
            for g in (1, 2, 3):
                pl.semaphore_signal(
                    sem,
                    inc=1,
                    device_id=((my_i + g) % N_DEV,),
                    device_id_type=pl.DeviceIdType.MESH,
                )
            pl.semaphore_wait(sem, 3)

    return pl.pallas_call(
        body,
        out_shape=jax.ShapeDtypeStruct((1, SQ, D_MODEL), jnp.float32),
        in_specs=[
            pl.BlockSpec(memory_space=pltpu.MemorySpace.VMEM),
            pl.BlockSpec(memory_space=pltpu.MemorySpace.VMEM),
            pl.BlockSpec(memory_space=pl.ANY),
            pl.BlockSpec(memory_space=pl.ANY),
            pl.BlockSpec(memory_space=pltpu.MemorySpace.VMEM),
        ],
        out_specs=pl.BlockSpec(memory_space=pltpu.MemorySpace.VMEM),
        scratch_shapes=[
            pltpu.VMEM((N_DEV, D_MODEL, DQ_PER), jnp.bfloat16),
            pltpu.VMEM((N_DEV, DQ_PER, D_MODEL), jnp.bfloat16),
            pltpu.VMEM((2, HQ_PER, SKV, DH), jnp.bfloat16),
            pltpu.VMEM((2, HQ_PER, SKV, DH), jnp.bfloat16),
            pltpu.SemaphoreType.DMA((N_DEV,)),
            pltpu.SemaphoreType.DMA((N_DEV,)),
            pltpu.SemaphoreType.DMA((N_DEV,)),
            pltpu.SemaphoreType.DMA((N_DEV,)),
            pltpu.SemaphoreType.DMA((2,)),
            pltpu.SemaphoreType.DMA((2,)),
        ],
        compiler_params=pltpu.CompilerParams(
            collective_id=0,
            vmem_limit_bytes=110 * 1024 * 1024,
        ),
    )(xb, wq, k_my, v_my, wo)


# device time: 333785 ns/iter; 1.3189x vs baseline; 1.1222x over previous
import functools

import jax
import jax.numpy as jnp
from jax import lax
from jax.experimental import pallas as pl
from jax.experimental.pallas import tpu as pltpu

N_DEV = 4
SQ = 2048
SKV = 2048
D_MODEL = 1024
HQ_PER = 8
DH = 128
DQ_PER = HQ_PER * DH
HKV = 32 * DH
SCALE = 0.08838834764831843
NEG = -30000.0
QT = 128
GW = 128
WIN = 384
KW = GW + WIN

ORDER = (0, 1, 3, 2)


def _softmax_ctx(s_biased, v):
    e = jnp.exp(s_biased)
    denom = jnp.sum(e, axis=1, keepdims=True)
    w = (e * (1.0 / denom)).astype(jnp.bfloat16)
    return lax.dot_general(
        w, v, (((1,), (0,)), ((), ())), preferred_element_type=jnp.float32
    ).astype(jnp.bfloat16)


def kernel(x, Wq, K_ext, V_ext, Wo):
    xb = x[0].astype(jnp.bfloat16)
    wq = Wq.astype(jnp.bfloat16)
    wo = Wo.astype(jnp.bfloat16)
    my = lax.axis_index("i")
    k_my = (
        lax.dynamic_index_in_dim(K_ext, my, 0, keepdims=False)
        .astype(jnp.bfloat16)
        .reshape(SKV, HKV)
    )
    v_my = (
        lax.dynamic_index_in_dim(V_ext, my, 0, keepdims=False)
        .astype(jnp.bfloat16)
        .reshape(SKV, HKV)
    )

    def body(
        x_ref,
        wq_ref,
        k_hbm,
        v_hbm,
        wo_ref,
        out_ref,
        wq_buf,
        wo_buf,
        k_scr,
        v_scr,
        sendq_sems,
        sendo_sems,
        recvq_sems,
        recvo_sems,
        k_sems,
        v_sems,
    ):
        my_i = lax.axis_index("i")

        barrier_sem = pltpu.get_barrier_semaphore()
        for g in (1, 2, 3):
            pl.semaphore_signal(
                barrier_sem,
                inc=1,
                device_id=((my_i + g) % N_DEV,),
                device_id_type=pl.DeviceIdType.MESH,
            )
        pl.semaphore_wait(barrier_sem, 3)

        wq_buf[my_i] = wq_ref[...]
        wo_buf[my_i] = wo_ref[...]
        sends = []
        for g in (1, 2, 3):
            for buf, ssems, rsems in (
                (wq_buf, sendq_sems, recvq_sems),
                (wo_buf, sendo_sems, recvo_sems),
            ):
                rdma = pltpu.make_async_remote_copy(
                    src_ref=buf.at[my_i],
                    dst_ref=buf.at[my_i],
                    send_sem=ssems.at[g],
                    recv_sem=rsems.at[4 - g],
                    device_id=((my_i + g) % N_DEV,),
                    device_id_type=pl.DeviceIdType.MESH,
                )
                rdma.start()
                sends.append(rdma)

        def kv_dma(j, slot):
            og = (my_i + ORDER[j]) % N_DEV
            cols = pl.ds(og * DQ_PER, DQ_PER)
            kop = pltpu.make_async_copy(
                k_hbm.at[:, cols], k_scr.at[slot], k_sems.at[slot]
            )
            vop = pltpu.make_async_copy(
                v_hbm.at[:, cols], v_scr.at[slot], v_sems.at[slot]
            )
            return kop, vop

        k0, v0 = kv_dma(0, 0)
        k0.start()
        v0.start()

        for j in range(N_DEV):
            d = ORDER[j]
            o = (my_i + d) % N_DEV
            slot = j % 2

            if j < N_DEV - 1:
                kn, vn = kv_dma(j + 1, 1 - slot)
                kn.start()
                vn.start()

            if d != 0:
                for buf, rsems in ((wq_buf, recvq_sems), (wo_buf, recvo_sems)):
                    recv = pltpu.make_async_remote_copy(
                        src_ref=buf.at[o],
                        dst_ref=buf.at[o],
                        send_sem=sendq_sems.at[0],
                        recv_sem=rsems.at[d],
                        device_id=(my_i,),
                        device_id_type=pl.DeviceIdType.MESH,
                    )
                    recv.wait_recv()
            kw_, vw_ = kv_dma(j, slot)
            kw_.wait()
            vw_.wait()

            x_t = x_ref[0:QT, :]
            q_t = lax.dot_general(
                x_t,
                wq_buf[o],
                (((1,), (0,)), ((), ())),
                preferred_element_type=jnp.float32,
            )
            q_t = (q_t * SCALE).astype(jnp.bfloat16)
            qi = lax.broadcasted_iota(jnp.int32, (QT, SKV), 0)
            ki = lax.broadcasted_iota(jnp.int32, (QT, SKV), 1)
            keep = (jnp.abs(qi - ki) <= 128) | (ki < 32) | (qi < 32)
            bias0 = jnp.where(keep, 0.0, NEG)
            ctx_parts = []
            for h in range(HQ_PER):
                hs = h * DH
                s = lax.dot_general(
                    q_t[:, hs : hs + DH],
                    k_scr[slot, :, hs : hs + DH],
                    (((1,), (1,)), ((), ())),
                    preferred_element_type=jnp.float32,
                )
                ctx_parts.append(
                    _softmax_ctx(s + bias0, v_scr[slot, :, hs : hs + DH])
                )
            ctx = jnp.concatenate(ctx_parts, axis=1)
            part = lax.dot_general(
                ctx,
                wo_buf[o],
                (((1,), (0,)), ((), ())),
                preferred_element_type=jnp.float32,
            )
            if j == 0:
                out_ref[0, 0:QT, :] = part
            else:
                out_ref[0, 0:QT, :] += part

            def tile_step(qt, _):
                r0 = qt * QT
                s0 = jnp.minimum(r0 - 128, SKV - WIN)
                x_tt = x_ref[pl.ds(r0, QT), :]
                q_tt = lax.dot_general(
                    x_tt,
                    wq_buf[o],
                    (((1,), (0,)), ((), ())),
                    preferred_element_type=jnp.float32,
                )
                q_tt = (q_tt * SCALE).astype(jnp.bfloat16)
                row = lax.broadcasted_iota(jnp.int32, (QT, KW), 0) + r0
                col = lax.broadcasted_iota(jnp.int32, (QT, KW), 1)
                in_glob = col < GW
                kiw = s0 + col - GW
                win_keep = (jnp.abs(row - kiw) <= 128) | (kiw < 32)
                keep_t = (in_glob & (col < 32) & (s0 >= GW)) | (
                    (~in_glob) & win_keep
                )
                bias_t = jnp.where(keep_t, 0.0, NEG)
                kcat = jnp.concatenate(
                    [k_scr[slot, 0:GW, :], k_scr[slot, pl.ds(s0, WIN), :]],
                    axis=0,
                )
                vcat = jnp.concatenate(
                    [v_scr[slot, 0:GW, :], v_scr[slot, pl.ds(s0, WIN), :]],
                    axis=0,
                )
                parts = []
                for h in range(HQ_PER):
                    hs = h * DH
                    s = lax.dot_general(
                        q_tt[:, hs : hs + DH],
                        kcat[:, hs : hs + DH],
                        (((1,), (1,)), ((), ())),
                        preferred_element_type=jnp.float32,
                    )
                    parts.append(_softmax_ctx(s + bias_t, vcat[:, hs : hs + DH]))
                ctx_t = jnp.concatenate(parts, axis=1)
                part_t = lax.dot_general(
                    ctx_t,
                    wo_buf[o],
                    (((1,), (0,)), ((), ())),
                    preferred_element_type=jnp.float32,
                )
                if j == 0:
                    out_ref[0, pl.ds(r0, QT), :] = part_t
                else:
                    out_ref[0, pl.ds(r0, QT), :] += part_t
                return 0

            lax.fori_loop(1, SQ // QT, tile_step, 0)

        for s_ in sends:
            s_.wait_send()

        @functools.partial(pl.run_scoped, sem=pltpu.SemaphoreType.REGULAR)
        def _(sem):
            for g in (1, 2, 3):
                pl.semaphore_signal(
                    sem,
                    inc=1,
                    device_id=((my_i + g) % N_DEV,),
                    device_id_type=pl.DeviceIdType.MESH,
                )
            pl.semaphore_wait(sem, 3)

    return pl.pallas_call(
        body,
        out_shape=jax.ShapeDtypeStruct((1, SQ, D_MODEL), jnp.float32),
        in_specs=[
            pl.BlockSpec(memory_space=pltpu.MemorySpace.VMEM),
            pl.BlockSpec(memory_space=pltpu.MemorySpace.VMEM),
            pl.BlockSpec(memory_space=pl.ANY),
            pl.BlockSpec(memory_space=pl.ANY),
            pl.BlockSpec(memory_space=pltpu.MemorySpace.VMEM),
        ],
        out_specs=pl.BlockSpec(memory_space=pltpu.MemorySpace.VMEM),
        scratch_shapes=[
            pltpu.VMEM((N_DEV, D_MODEL, DQ_PER), jnp.bfloat16),
            pltpu.VMEM((N_DEV, DQ_PER, D_MODEL), jnp.bfloat16),
            pltpu.VMEM((2, SKV, DQ_PER), jnp.bfloat16),
            pltpu.VMEM((2, SKV, DQ_PER), jnp.bfloat16),
            pltpu.SemaphoreType.DMA((N_DEV,)),
            pltpu.SemaphoreType.DMA((N_DEV,)),
            pltpu.SemaphoreType.DMA((N_DEV,)),
            pltpu.SemaphoreType.DMA((N_DEV,)),
            pltpu.SemaphoreType.DMA((2,)),
            pltpu.SemaphoreType.DMA((2,)),
        ],
        compiler_params=pltpu.CompilerParams(
            collective_id=0,
            vmem_limit_bytes=110 * 1024 * 1024,
        ),
    )(xb, wq, k_my, v_my, wo)


# device time: 298030 ns/iter; 1.4771x vs baseline; 1.1200x over previous
import functools

import jax
import jax.numpy as jnp
from jax import lax
from jax.experimental import pallas as pl
from jax.experimental.pallas import tpu as pltpu

N_DEV = 4
SQ = 2048
SKV = 2048
D_MODEL = 1024
HQ_PER = 8
DH = 128
DQ_PER = HQ_PER * DH
HKV = 32 * DH
SCALE = 0.08838834764831843
NEG = -30000.0
QT = 256
GW = 128
WIN = QT + 256
KW = GW + WIN

ORDER = (0, 1, 3, 2)


def _softmax_ctx(s_biased, v):
    e = jnp.exp(s_biased)
    denom = jnp.sum(e, axis=1, keepdims=True)
    w = (e * (1.0 / denom)).astype(jnp.bfloat16)
    return lax.dot_general(
        w, v, (((1,), (0,)), ((), ())), preferred_element_type=jnp.float32
    ).astype(jnp.bfloat16)


def kernel(x, Wq, K_ext, V_ext, Wo):
    xb = x[0].astype(jnp.bfloat16)
    wq = Wq.astype(jnp.bfloat16)
    wo = Wo.astype(jnp.bfloat16)
    my = lax.axis_index("i")
    k_my = (
        lax.dynamic_index_in_dim(K_ext, my, 0, keepdims=False)
        .astype(jnp.bfloat16)
        .reshape(SKV, HKV)
    )
    v_my = (
        lax.dynamic_index_in_dim(V_ext, my, 0, keepdims=False)
        .astype(jnp.bfloat16)
        .reshape(SKV, HKV)
    )

    def body(
        x_ref,
        wq_ref,
        k_hbm,
        v_hbm,
        wo_ref,
        out_ref,
        wq_buf,
        wo_buf,
        k_scr,
        v_scr,
        sendq_sems,
        sendo_sems,
        recvq_sems,
        recvo_sems,
        k_sems,
        v_sems,
    ):
        my_i = lax.axis_index("i")

        barrier_sem = pltpu.get_barrier_semaphore()
        for g in (1, 2, 3):
            pl.semaphore_signal(
                barrier_sem,
                inc=1,
                device_id=((my_i + g) % N_DEV,),
                device_id_type=pl.DeviceIdType.MESH,
            )
        pl.semaphore_wait(barrier_sem, 3)

        wq_buf[my_i] = wq_ref[...]
        wo_buf[my_i] = wo_ref[...]
        sends = []
        for g in (1, 2, 3):
            for buf, ssems, rsems in (
                (wq_buf, sendq_sems, recvq_sems),
                (wo_buf, sendo_sems, recvo_sems),
            ):
                rdma = pltpu.make_async_remote_copy(
                    src_ref=buf.at[my_i],
                    dst_ref=buf.at[my_i],
                    send_sem=ssems.at[g],
                    recv_sem=rsems.at[4 - g],
                    device_id=((my_i + g) % N_DEV,),
                    device_id_type=pl.DeviceIdType.MESH,
                )
                rdma.start()
                sends.append(rdma)

        def kv_dma(j, slot):
            og = (my_i + ORDER[j]) % N_DEV
            cols = pl.ds(og * DQ_PER, DQ_PER)
            kop = pltpu.make_async_copy(
                k_hbm.at[:, cols], k_scr.at[slot], k_sems.at[slot]
            )
            vop = pltpu.make_async_copy(
                v_hbm.at[:, cols], v_scr.at[slot], v_sems.at[slot]
            )
            return kop, vop

        k0, v0 = kv_dma(0, 0)
        k0.start()
        v0.start()

        for j in range(N_DEV):
            d = ORDER[j]
            o = (my_i + d) % N_DEV
            slot = j % 2

            if j < N_DEV - 1:
                kn, vn = kv_dma(j + 1, 1 - slot)
                kn.start()
                vn.start()

            if d != 0:
                for buf, rsems in ((wq_buf, recvq_sems), (wo_buf, recvo_sems)):
                    recv = pltpu.make_async_remote_copy(
                        src_ref=buf.at[o],
                        dst_ref=buf.at[o],
                        send_sem=sendq_sems.at[0],
                        recv_sem=rsems.at[d],
                        device_id=(my_i,),
                        device_id_type=pl.DeviceIdType.MESH,
                    )
                    recv.wait_recv()
            kw_, vw_ = kv_dma(j, slot)
            kw_.wait()
            vw_.wait()

            x_t = x_ref[0:QT, :]
            q_t = lax.dot_general(
                x_t,
                wq_buf[o],
                (((1,), (0,)), ((), ())),
                preferred_element_type=jnp.float32,
            )
            q_t = (q_t * SCALE).astype(jnp.bfloat16)
            qi = lax.broadcasted_iota(jnp.int32, (QT, SKV), 0)
            ki = lax.broadcasted_iota(jnp.int32, (QT, SKV), 1)
            keep = (jnp.abs(qi - ki) <= 128) | (ki < 32) | (qi < 32)
            bias0 = jnp.where(keep, 0.0, NEG)
            ctx_parts = []
            for h in range(HQ_PER):
                hs = h * DH
                s = lax.dot_general(
                    q_t[:, hs : hs + DH],
                    k_scr[slot, :, hs : hs + DH],
                    (((1,), (1,)), ((), ())),
                    preferred_element_type=jnp.float32,
                )
                ctx_parts.append(
                    _softmax_ctx(s + bias0, v_scr[slot, :, hs : hs + DH])
                )
            ctx = jnp.concatenate(ctx_parts, axis=1)
            part = lax.dot_general(
                ctx,
                wo_buf[o],
                (((1,), (0,)), ((), ())),
                preferred_element_type=jnp.float32,
            )
            if j == 0:
                out_ref[0, 0:QT, :] = part
            else:
                out_ref[0, 0:QT, :] += part

            def tile_step(qt, _):
                r0 = qt * QT
                s0 = jnp.minimum(r0 - 128, SKV - WIN)
                x_tt = x_ref[pl.ds(r0, QT), :]
                q_tt = lax.dot_general(
                    x_tt,
                    wq_buf[o],
                    (((1,), (0,)), ((), ())),
                    preferred_element_type=jnp.float32,
                )
                q_tt = (q_tt * SCALE).astype(jnp.bfloat16)
                row = lax.broadcasted_iota(jnp.int32, (QT, KW), 0) + r0
                col = lax.broadcasted_iota(jnp.int32, (QT, KW), 1)
                in_glob = col < GW
                kiw = s0 + col - GW
                win_keep = (jnp.abs(row - kiw) <= 128) | (kiw < 32)
                keep_t = (in_glob & (col < 32) & (s0 >= GW)) | (
                    (~in_glob) & win_keep
                )
                bias_t = jnp.where(keep_t, 0.0, NEG)
                kcat = jnp.concatenate(
                    [k_scr[slot, 0:GW, :], k_scr[slot, pl.ds(s0, WIN), :]],
                    axis=0,
                )
                vcat = jnp.concatenate(
                    [v_scr[slot, 0:GW, :], v_scr[slot, pl.ds(s0, WIN), :]],
                    axis=0,
                )
                parts = []
                for h in range(HQ_PER):
                    hs = h * DH
                    s = lax.dot_general(
                        q_tt[:, hs : hs + DH],
                        kcat[:, hs : hs + DH],
                        (((1,), (1,)), ((), ())),
                        preferred_element_type=jnp.float32,
                    )
                    parts.append(_softmax_ctx(s + bias_t, vcat[:, hs : hs + DH]))
                ctx_t = jnp.concatenate(parts, axis=1)
                part_t = lax.dot_general(
                    ctx_t,
                    wo_buf[o],
                    (((1,), (0,)), ((), ())),
                    preferred_element_type=jnp.float32,
                )
                if j == 0:
                    out_ref[0, pl.ds(r0, QT), :] = part_t
                else:
                    out_ref[0, pl.ds(r0, QT), :] += part_t
                return 0

            lax.fori_loop(1, SQ // QT, tile_step, 0)

        for s_ in sends:
            s_.wait_send()

        @functools.partial(pl.run_scoped, sem=pltpu.SemaphoreType.REGULAR)
        def _(sem):
            for g in (1, 2, 3):
                pl.semaphore_signal(
                    sem,
                    inc=1,
                    device_id=((my_i + g) % N_DEV,),
                    device_id_type=pl.DeviceIdType.MESH,
                )
            pl.semaphore_wait(sem, 3)

    return pl.pallas_call(
        body,
        out_shape=jax.ShapeDtypeStruct((1, SQ, D_MODEL), jnp.float32),
        in_specs=[
            pl.BlockSpec(memory_space=pltpu.MemorySpace.VMEM),
            pl.BlockSpec(memory_space=pltpu.MemorySpace.VMEM),
            pl.BlockSpec(memory_space=pl.ANY),
            pl.BlockSpec(memory_space=pl.ANY),
            pl.BlockSpec(memory_space=pltpu.MemorySpace.VMEM),
        ],
        out_specs=pl.BlockSpec(memory_space=pltpu.MemorySpace.VMEM),
        scratch_shapes=[
            pltpu.VMEM((N_DEV, D_MODEL, DQ_PER), jnp.bfloat16),
            pltpu.VMEM((N_DEV, DQ_PER, D_MODEL), jnp.bfloat16),
            pltpu.VMEM((2, SKV, DQ_PER), jnp.bfloat16),
            pltpu.VMEM((2, SKV, DQ_PER), jnp.bfloat16),
            pltpu.SemaphoreType.DMA((N_DEV,)),
            pltpu.SemaphoreType.DMA((N_DEV,)),
            pltpu.SemaphoreType.DMA((N_DEV,)),
            pltpu.SemaphoreType.DMA((N_DEV,)),
            pltpu.SemaphoreType.DMA((2,)),
            pltpu.SemaphoreType.DMA((2,)),
        ],
        compiler_params=pltpu.CompilerParams(
            collective_id=0,
            vmem_limit_bytes=110 * 1024 * 1024,
        ),
    )(xb, wq, k_my, v_my, wo)
